# baseline (speedup 1.0000x reference)
import functools

import numpy as np
import jax
import jax.numpy as jnp
from jax import lax
from jax.experimental import pallas as pl
from jax.experimental.pallas import tpu as pltpu
from jax.experimental.pallas import tpu_sc as plsc

_SAMPLE_SIZE = 25
_NCOLS = _SAMPLE_SIZE + 1
_NODES = 50000
_BATCH = 4096
_ADJ_COLS = 65
_FEAT = 256
_NW = 32
_ROWS_PER_W = _BATCH // _NW
_IDS_PER_W = _ROWS_PER_W * _NCOLS
_CHUNK = 128
_NCHUNK = _IDS_PER_W // _CHUNK

_PERM25 = (35, 45, 31, 63, 7, 4, 29, 44, 16, 58, 37, 19, 61, 2, 34,
           5, 30, 42, 3, 39, 56, 22, 6, 54, 18)
_COLS = np.array([0] + [p + 1 for p in _PERM25], dtype=np.int32)

_cache = {}


def _flat_tab():
    if "tab" not in _cache:
        w, k, bi = np.meshgrid(np.arange(_NW), np.arange(_NCOLS),
                               np.arange(_ROWS_PER_W), indexing="ij")
        _cache["tab"] = ((w * _ROWS_PER_W + bi) * _ADJ_COLS
                         + _COLS[k]).reshape(-1).astype(np.int32)
    return _cache["tab"]


def _build_gather():
    if "call" in _cache:
        return _cache["call"]

    mesh = plsc.VectorSubcoreMesh(core_axis_name="c", subcore_axis_name="s")

    @functools.partial(
        pl.kernel,
        mesh=mesh,
        out_type=jax.ShapeDtypeStruct((_NCOLS, _BATCH, _FEAT), jnp.float32),
        scratch_types=[
            pltpu.VMEM((_IDS_PER_W,), jnp.int32),
            pltpu.VMEM((_IDS_PER_W,), jnp.int32),
            pltpu.VMEM((_CHUNK, _FEAT), jnp.float32),
            pltpu.VMEM((_CHUNK, _FEAT), jnp.float32),
            pltpu.SemaphoreType.DMA,
            pltpu.SemaphoreType.DMA,
            pltpu.SemaphoreType.DMA,
            pltpu.SemaphoreType.DMA,
            pltpu.SemaphoreType.DMA,
        ],
    )
    def _impl(x_hbm, adj_hbm, tab_hbm, out_hbm,
              tab_v, ids_v, buf0, buf1, asem, g0, g1, w0, w1):
        wid = lax.axis_index("s") * 2 + lax.axis_index("c")
        b0 = wid * _ROWS_PER_W

        pltpu.sync_copy(tab_hbm.at[pl.ds(wid * _IDS_PER_W, _IDS_PER_W)],
                        tab_v)

        for c in range(_NCHUNK):
            sl = pl.ds(c * _CHUNK, _CHUNK)
            pltpu.make_async_copy(
                adj_hbm.at[tab_v.at[sl]], ids_v.at[sl], asem).start()
        pltpu.make_async_copy(adj_hbm.at[tab_v], ids_v, asem).wait()

        buf = (buf0, buf1)
        gsem = (g0, g1)
        wsem = (w0, w1)

        def g_copy(c, b):
            return pltpu.make_async_copy(
                x_hbm.at[ids_v.at[pl.ds(c * _CHUNK, _CHUNK)]],
                buf[b], gsem[b])

        def w_copy(c, b):
            return pltpu.make_async_copy(
                buf[b],
                out_hbm.at[c, pl.ds(b0, _ROWS_PER_W)], wsem[b])

        g_copy(0, 0).start()
        g_copy(1, 1).start()

        for c in range(_NCHUNK):
            b = c & 1
            g_copy(c, b).wait()
            w_copy(c, b).start()
            if 1 <= c and c + 1 < _NCHUNK:
                pb = (c - 1) & 1
                w_copy(c - 1, pb).wait()
                g_copy(c + 1, pb).start()

        w_copy(_NCHUNK - 2, 0).wait()
        w_copy(_NCHUNK - 1, 1).wait()

    _cache["call"] = _impl
    return _impl


_KBLK = 13


def _fmt_body(g_ref, o_ref):
    for kk in range(_KBLK):
        blk = g_ref[kk]
        o_ref[pl.ds(kk * 4, 4)] = jnp.swapaxes(blk, 0, 1).reshape(
            4, 8, 1, 8, _ROWS_PER_W)


def _build_format():
    if "fmt" not in _cache:
        _cache["fmt"] = pl.pallas_call(
            _fmt_body,
            grid=(_NCOLS // _KBLK, _BATCH // _ROWS_PER_W),
            in_specs=[pl.BlockSpec((_KBLK, _ROWS_PER_W, _FEAT),
                                   lambda k, t: (k, t, 0))],
            out_specs=pl.BlockSpec((4 * _KBLK, 8, 1, 8, _ROWS_PER_W),
                                   lambda k, t: (k, 0, t, 0, 0)),
            out_shape=jax.ShapeDtypeStruct(
                (_NCOLS * 4, 8, _BATCH // _ROWS_PER_W, 8, _ROWS_PER_W),
                jnp.float32),
        )
    return _cache["fmt"]


_NBLK = 1024


def _xpose_body(xt_ref, x2_ref):
    x2_ref[...] = jnp.swapaxes(xt_ref[...], 0, 1)


def _build_xpose():
    if "xp" not in _cache:
        _cache["xp"] = pl.pallas_call(
            _xpose_body,
            grid=(pl.cdiv(_NODES, _NBLK),),
            in_specs=[pl.BlockSpec((_FEAT, _NBLK), lambda i: (0, i))],
            out_specs=pl.BlockSpec((_NBLK, _FEAT), lambda i: (i, 0)),
            out_shape=jax.ShapeDtypeStruct((_NODES, _FEAT), jnp.float32),
        )
    return _cache["xp"]


def kernel(x, adj_input):
    x2 = _build_xpose()(x.reshape(_NODES, _FEAT).T)
    adj = adj_input.astype(jnp.int32).reshape(_BATCH * _ADJ_COLS)
    tab = jnp.asarray(_flat_tab())
    g = _build_gather()(x2, adj, tab)
    o5 = _build_format()(g)
    return jnp.transpose(o5, (2, 4, 0, 1, 3)).reshape(_BATCH, _NCOLS * 4, 8, 8)


# speedup vs baseline: 1.0532x; 1.0532x over previous
import functools

import numpy as np
import jax
import jax.numpy as jnp
from jax import lax
from jax.experimental import pallas as pl
from jax.experimental.pallas import tpu as pltpu
from jax.experimental.pallas import tpu_sc as plsc

_SAMPLE_SIZE = 25
_NCOLS = _SAMPLE_SIZE + 1
_NODES = 50000
_BATCH = 4096
_ADJ_COLS = 65
_FEAT = 256
_NW = 32
_ROWS_PER_W = _BATCH // _NW
_IDS_PER_W = _ROWS_PER_W * _NCOLS
_CHUNK = 128
_NCHUNK = _IDS_PER_W // _CHUNK

_PERM25 = (35, 45, 31, 63, 7, 4, 29, 44, 16, 58, 37, 19, 61, 2, 34,
           5, 30, 42, 3, 39, 56, 22, 6, 54, 18)
_COLS = np.array([0] + [p + 1 for p in _PERM25], dtype=np.int32)

_cache = {}


def _flat_tab():
    if "tab" not in _cache:
        w, k, bi = np.meshgrid(np.arange(_NW), np.arange(_NCOLS),
                               np.arange(_ROWS_PER_W), indexing="ij")
        _cache["tab"] = (_COLS[k] * _BATCH
                         + (w * _ROWS_PER_W + bi)).reshape(-1).astype(np.int32)
    return _cache["tab"]


def _build_gather():
    if "call" in _cache:
        return _cache["call"]

    mesh = plsc.VectorSubcoreMesh(core_axis_name="c", subcore_axis_name="s")

    @functools.partial(
        pl.kernel,
        mesh=mesh,
        out_type=jax.ShapeDtypeStruct((_NCOLS, _BATCH, _FEAT), jnp.float32),
        scratch_types=[
            pltpu.VMEM((_IDS_PER_W,), jnp.int32),
            pltpu.VMEM((_IDS_PER_W,), jnp.int32),
            pltpu.VMEM((_CHUNK, _FEAT), jnp.float32),
            pltpu.VMEM((_CHUNK, _FEAT), jnp.float32),
            pltpu.VMEM((_CHUNK, _FEAT), jnp.float32),
            pltpu.SemaphoreType.DMA,
            pltpu.SemaphoreType.DMA,
            pltpu.SemaphoreType.DMA,
            pltpu.SemaphoreType.DMA,
            pltpu.SemaphoreType.DMA,
            pltpu.SemaphoreType.DMA,
            pltpu.SemaphoreType.DMA,
        ],
    )
    def _impl(x_hbm, adj_hbm, tab_hbm, out_hbm,
              tab_v, ids_v, buf0, buf1, buf2,
              asem, g0, g1, g2, w0, w1, w2):
        wid = lax.axis_index("s") * 2 + lax.axis_index("c")
        b0 = wid * _ROWS_PER_W

        pltpu.sync_copy(tab_hbm.at[pl.ds(wid * _IDS_PER_W, _IDS_PER_W)],
                        tab_v)

        for c in range(_NCHUNK):
            sl = pl.ds(c * _CHUNK, _CHUNK)
            pltpu.make_async_copy(
                adj_hbm.at[tab_v.at[sl]], ids_v.at[sl], asem).start()
        pltpu.make_async_copy(adj_hbm.at[tab_v], ids_v, asem).wait()

        buf = (buf0, buf1, buf2)
        gsem = (g0, g1, g2)
        wsem = (w0, w1, w2)

        def g_copy(c, b):
            return pltpu.make_async_copy(
                x_hbm.at[ids_v.at[pl.ds(c * _CHUNK, _CHUNK)]],
                buf[b], gsem[b])

        def w_copy(c, b):
            return pltpu.make_async_copy(
                buf[b],
                out_hbm.at[c, pl.ds(b0, _ROWS_PER_W)], wsem[b])

        g_copy(0, 0).start()
        g_copy(1, 1).start()

        for c in range(_NCHUNK):
            b = c % 3
            g_copy(c, b).wait()
            w_copy(c, b).start()
            if c + 2 < _NCHUNK:
                if c >= 1:
                    w_copy(c - 1, (c - 1) % 3).wait()
                g_copy(c + 2, (c + 2) % 3).start()

        for c in range(_NCHUNK - 3, _NCHUNK):
            w_copy(c, c % 3).wait()

    _cache["call"] = _impl
    return _impl


_KBLK = 13


def _fmt_body(g_ref, o_ref):
    for kk in range(_KBLK):
        blk = g_ref[kk]
        o_ref[pl.ds(kk * 4, 4)] = jnp.swapaxes(blk, 0, 1).reshape(
            4, 8, 1, 8, _ROWS_PER_W)


def _build_format():
    if "fmt" not in _cache:
        _cache["fmt"] = pl.pallas_call(
            _fmt_body,
            grid=(_NCOLS // _KBLK, _BATCH // _ROWS_PER_W),
            in_specs=[pl.BlockSpec((_KBLK, _ROWS_PER_W, _FEAT),
                                   lambda k, t: (k, t, 0))],
            out_specs=pl.BlockSpec((4 * _KBLK, 8, 1, 8, _ROWS_PER_W),
                                   lambda k, t: (k, 0, t, 0, 0)),
            out_shape=jax.ShapeDtypeStruct(
                (_NCOLS * 4, 8, _BATCH // _ROWS_PER_W, 8, _ROWS_PER_W),
                jnp.float32),
        )
    return _cache["fmt"]


_NBLK = 1024


def _xpose_body(xt_ref, x2_ref):
    x2_ref[...] = jnp.swapaxes(xt_ref[...], 0, 1)


def _build_xpose():
    if "xp" not in _cache:
        _cache["xp"] = pl.pallas_call(
            _xpose_body,
            grid=(pl.cdiv(_NODES, _NBLK),),
            in_specs=[pl.BlockSpec((_FEAT, _NBLK), lambda i: (0, i))],
            out_specs=pl.BlockSpec((_NBLK, _FEAT), lambda i: (i, 0)),
            out_shape=jax.ShapeDtypeStruct((_NODES, _FEAT), jnp.float32),
        )
    return _cache["xp"]


def kernel(x, adj_input):
    x2 = x.reshape(_NODES, _FEAT)
    adj = adj_input.astype(jnp.int32).T.reshape(_ADJ_COLS * _BATCH)
    tab = jnp.asarray(_flat_tab())
    g = _build_gather()(x2, adj, tab)
    o5 = _build_format()(g)
    return jnp.transpose(o5, (2, 4, 0, 1, 3)).reshape(_BATCH, _NCOLS * 4, 8, 8)


# speedup vs baseline: 1.0840x; 1.0293x over previous
import functools

import numpy as np
import jax
import jax.numpy as jnp
from jax import lax
from jax.experimental import pallas as pl
from jax.experimental.pallas import tpu as pltpu
from jax.experimental.pallas import tpu_sc as plsc

_SAMPLE_SIZE = 25
_NCOLS = _SAMPLE_SIZE + 1
_NODES = 50000
_BATCH = 4096
_ADJ_COLS = 65
_FEAT = 256
_NW = 32
_ROWS_PER_W = _BATCH // _NW
_IDS_PER_W = _ROWS_PER_W * _NCOLS
_CHUNK = 128
_NCHUNK = _IDS_PER_W // _CHUNK

_PERM25 = (35, 45, 31, 63, 7, 4, 29, 44, 16, 58, 37, 19, 61, 2, 34,
           5, 30, 42, 3, 39, 56, 22, 6, 54, 18)
_COLS = np.array([0] + [p + 1 for p in _PERM25], dtype=np.int32)

_cache = {}


def _flat_tab():
    if "tab" not in _cache:
        w, k, bi = np.meshgrid(np.arange(_NW), np.arange(_NCOLS),
                               np.arange(_ROWS_PER_W), indexing="ij")
        _cache["tab"] = (_COLS[k] * _BATCH
                         + (w * _ROWS_PER_W + bi)).reshape(-1).astype(np.int32)
    return _cache["tab"]


def _build_gather(k0, nk):
    key = ("call", k0, nk)
    if key in _cache:
        return _cache[key]

    ids_n = nk * _CHUNK
    mesh = plsc.VectorSubcoreMesh(core_axis_name="c", subcore_axis_name="s")

    @functools.partial(
        pl.kernel,
        mesh=mesh,
        out_type=jax.ShapeDtypeStruct((nk, _BATCH, _FEAT), jnp.float32),
        scratch_types=[
            pltpu.VMEM((ids_n,), jnp.int32),
            pltpu.VMEM((ids_n,), jnp.int32),
            pltpu.VMEM((_CHUNK, _FEAT), jnp.float32),
            pltpu.VMEM((_CHUNK, _FEAT), jnp.float32),
            pltpu.VMEM((_CHUNK, _FEAT), jnp.float32),
            pltpu.SemaphoreType.DMA,
            pltpu.SemaphoreType.DMA,
            pltpu.SemaphoreType.DMA,
            pltpu.SemaphoreType.DMA,
            pltpu.SemaphoreType.DMA,
            pltpu.SemaphoreType.DMA,
            pltpu.SemaphoreType.DMA,
        ],
    )
    def _impl(x_hbm, adj_hbm, tab_hbm, out_hbm,
              tab_v, ids_v, buf0, buf1, buf2,
              asem, g0, g1, g2, w0, w1, w2):
        wid = lax.axis_index("s") * 2 + lax.axis_index("c")
        b0 = wid * _ROWS_PER_W

        pltpu.sync_copy(
            tab_hbm.at[pl.ds(wid * _IDS_PER_W + k0 * _CHUNK, ids_n)], tab_v)

        for c in range(nk):
            sl = pl.ds(c * _CHUNK, _CHUNK)
            pltpu.make_async_copy(
                adj_hbm.at[tab_v.at[sl]], ids_v.at[sl], asem).start()
        pltpu.make_async_copy(adj_hbm.at[tab_v], ids_v, asem).wait()

        buf = (buf0, buf1, buf2)
        gsem = (g0, g1, g2)
        wsem = (w0, w1, w2)

        def g_copy(c, b):
            return pltpu.make_async_copy(
                x_hbm.at[ids_v.at[pl.ds(c * _CHUNK, _CHUNK)]],
                buf[b], gsem[b])

        def w_copy(c, b):
            return pltpu.make_async_copy(
                buf[b],
                out_hbm.at[c, pl.ds(b0, _ROWS_PER_W)], wsem[b])

        g_copy(0, 0).start()
        g_copy(1, 1).start()

        for c in range(nk):
            b = c % 3
            g_copy(c, b).wait()
            w_copy(c, b).start()
            if c + 2 < nk:
                if c >= 1:
                    w_copy(c - 1, (c - 1) % 3).wait()
                g_copy(c + 2, (c + 2) % 3).start()

        for c in range(max(nk - 3, 0), nk):
            w_copy(c, c % 3).wait()

    _cache[key] = _impl
    return _impl


_KBLK = 13


def _fmt_body(g_ref, o_ref):
    for kk in range(_KBLK):
        blk = g_ref[kk]
        o_ref[pl.ds(kk * 4, 4)] = jnp.swapaxes(blk, 0, 1).reshape(
            4, 8, 1, 8, _ROWS_PER_W)


def _fmt_body_alias(g_ref, o5in_ref, o_ref):
    del o5in_ref
    _fmt_body(g_ref, o_ref)


def _build_format(part):
    key = ("fmt", part)
    if key not in _cache:
        in_specs = [pl.BlockSpec((_KBLK, _ROWS_PER_W, _FEAT),
                                 lambda k, t: (k, t, 0))]
        kwargs = {}
        body = _fmt_body
        if part > 0:
            in_specs.append(pl.BlockSpec(memory_space=pl.ANY))
            kwargs["input_output_aliases"] = {1: 0}
            body = _fmt_body_alias
        _cache[key] = pl.pallas_call(
            body,
            grid=(1, _BATCH // _ROWS_PER_W),
            in_specs=in_specs,
            out_specs=pl.BlockSpec((4 * _KBLK, 8, 1, 8, _ROWS_PER_W),
                                   lambda k, t: (k + part, 0, t, 0, 0)),
            out_shape=jax.ShapeDtypeStruct(
                (_NCOLS * 4, 8, _BATCH // _ROWS_PER_W, 8, _ROWS_PER_W),
                jnp.float32),
            **kwargs,
        )
    return _cache[key]


def kernel(x, adj_input):
    x2 = x.reshape(_NODES, _FEAT)
    adj = adj_input.astype(jnp.int32).T.reshape(_ADJ_COLS * _BATCH)
    tab = jnp.asarray(_flat_tab())
    g0 = _build_gather(0, _KBLK)(x2, adj, tab)
    g1 = _build_gather(_KBLK, _KBLK)(x2, adj, tab)
    o5a = _build_format(0)(g0)
    o5 = _build_format(1)(g1, o5a)
    return jnp.transpose(o5, (2, 4, 0, 1, 3)).reshape(_BATCH, _NCOLS * 4, 8, 8)


# speedup vs baseline: 1.0871x; 1.0029x over previous
import functools

import numpy as np
import jax
import jax.numpy as jnp
from jax import lax
from jax.experimental import pallas as pl
from jax.experimental.pallas import tpu as pltpu
from jax.experimental.pallas import tpu_sc as plsc

_SAMPLE_SIZE = 25
_NCOLS = _SAMPLE_SIZE + 1
_NODES = 50000
_BATCH = 4096
_ADJ_COLS = 65
_FEAT = 256
_NW = 32
_ROWS_PER_W = _BATCH // _NW
_IDS_PER_W = _ROWS_PER_W * _NCOLS
_CHUNK = 128
_NCHUNK = _IDS_PER_W // _CHUNK

_PERM25 = (35, 45, 31, 63, 7, 4, 29, 44, 16, 58, 37, 19, 61, 2, 34,
           5, 30, 42, 3, 39, 56, 22, 6, 54, 18)
_COLS = np.array([0] + [p + 1 for p in _PERM25], dtype=np.int32)

_cache = {}


def _flat_tab():
    if "tab" not in _cache:
        w, k, bi = np.meshgrid(np.arange(_NW), np.arange(_NCOLS),
                               np.arange(_ROWS_PER_W), indexing="ij")
        _cache["tab"] = (_COLS[k] * _BATCH
                         + (w * _ROWS_PER_W + bi)).reshape(-1).astype(np.int32)
    return _cache["tab"]


def _build_ids():
    if "ids" in _cache:
        return _cache["ids"]

    mesh = plsc.VectorSubcoreMesh(core_axis_name="c", subcore_axis_name="s")

    @functools.partial(
        pl.kernel,
        mesh=mesh,
        out_type=jax.ShapeDtypeStruct((_NW * _IDS_PER_W,), jnp.int32),
        scratch_types=[
            pltpu.VMEM((_IDS_PER_W,), jnp.int32),
            pltpu.VMEM((_IDS_PER_W,), jnp.int32),
            pltpu.SemaphoreType.DMA,
        ],
    )
    def _impl(adj_hbm, tab_hbm, out_hbm, tab_v, ids_v, asem):
        wid = lax.axis_index("s") * 2 + lax.axis_index("c")
        base = wid * _IDS_PER_W
        pltpu.sync_copy(tab_hbm.at[pl.ds(base, _IDS_PER_W)], tab_v)
        for c in range(_NCHUNK):
            sl = pl.ds(c * _CHUNK, _CHUNK)
            pltpu.make_async_copy(
                adj_hbm.at[tab_v.at[sl]], ids_v.at[sl], asem).start()
        pltpu.make_async_copy(adj_hbm.at[tab_v], ids_v, asem).wait()
        pltpu.sync_copy(ids_v, out_hbm.at[pl.ds(base, _IDS_PER_W)])

    _cache["ids"] = _impl
    return _impl


def _build_gather(k0, nk):
    key = ("call", k0, nk)
    if key in _cache:
        return _cache[key]

    ids_n = nk * _CHUNK
    mesh = plsc.VectorSubcoreMesh(core_axis_name="c", subcore_axis_name="s")

    @functools.partial(
        pl.kernel,
        mesh=mesh,
        out_type=jax.ShapeDtypeStruct((nk, _BATCH, _FEAT), jnp.float32),
        scratch_types=[
            pltpu.VMEM((ids_n,), jnp.int32),
            pltpu.VMEM((_CHUNK, _FEAT), jnp.float32),
            pltpu.VMEM((_CHUNK, _FEAT), jnp.float32),
            pltpu.VMEM((_CHUNK, _FEAT), jnp.float32),
            pltpu.SemaphoreType.DMA,
            pltpu.SemaphoreType.DMA,
            pltpu.SemaphoreType.DMA,
            pltpu.SemaphoreType.DMA,
            pltpu.SemaphoreType.DMA,
            pltpu.SemaphoreType.DMA,
        ],
    )
    def _impl(x_hbm, allids_hbm, out_hbm,
              ids_v, buf0, buf1, buf2,
              g0, g1, g2, w0, w1, w2):
        wid = lax.axis_index("s") * 2 + lax.axis_index("c")
        b0 = wid * _ROWS_PER_W

        pltpu.sync_copy(
            allids_hbm.at[pl.ds(wid * _IDS_PER_W + k0 * _CHUNK, ids_n)],
            ids_v)

        buf = (buf0, buf1, buf2)
        gsem = (g0, g1, g2)
        wsem = (w0, w1, w2)

        def g_copy(c, b):
            return pltpu.make_async_copy(
                x_hbm.at[ids_v.at[pl.ds(c * _CHUNK, _CHUNK)]],
                buf[b], gsem[b])

        def w_copy(c, b):
            return pltpu.make_async_copy(
                buf[b],
                out_hbm.at[c, pl.ds(b0, _ROWS_PER_W)], wsem[b])

        g_copy(0, 0).start()
        g_copy(1, 1).start()

        for c in range(nk):
            b = c % 3
            g_copy(c, b).wait()
            w_copy(c, b).start()
            if c + 2 < nk:
                if c >= 1:
                    w_copy(c - 1, (c - 1) % 3).wait()
                g_copy(c + 2, (c + 2) % 3).start()

        for c in range(max(nk - 3, 0), nk):
            w_copy(c, c % 3).wait()

    _cache[key] = _impl
    return _impl


_KBLK = 13


def _fmt_body(g_ref, o_ref):
    for kk in range(_KBLK):
        blk = g_ref[kk]
        o_ref[pl.ds(kk * 4, 4)] = jnp.swapaxes(blk, 0, 1).reshape(
            4, 8, 1, 8, _ROWS_PER_W)


def _fmt_body_alias(g_ref, o5in_ref, o_ref):
    del o5in_ref
    _fmt_body(g_ref, o_ref)


def _build_format(part):
    key = ("fmt", part)
    if key not in _cache:
        in_specs = [pl.BlockSpec((_KBLK, _ROWS_PER_W, _FEAT),
                                 lambda k, t: (k, t, 0))]
        kwargs = {}
        body = _fmt_body
        if part > 0:
            in_specs.append(pl.BlockSpec(memory_space=pl.ANY))
            kwargs["input_output_aliases"] = {1: 0}
            body = _fmt_body_alias
        _cache[key] = pl.pallas_call(
            body,
            grid=(1, _BATCH // _ROWS_PER_W),
            in_specs=in_specs,
            out_specs=pl.BlockSpec((4 * _KBLK, 8, 1, 8, _ROWS_PER_W),
                                   lambda k, t: (k + part, 0, t, 0, 0)),
            out_shape=jax.ShapeDtypeStruct(
                (_NCOLS * 4, 8, _BATCH // _ROWS_PER_W, 8, _ROWS_PER_W),
                jnp.float32),
            **kwargs,
        )
    return _cache[key]


def kernel(x, adj_input):
    x2 = x.reshape(_NODES, _FEAT)
    adj = adj_input.astype(jnp.int32).T.reshape(_ADJ_COLS * _BATCH)
    tab = jnp.asarray(_flat_tab())
    ids = _build_ids()(adj, tab)
    g0 = _build_gather(0, _KBLK)(x2, ids)
    g1 = _build_gather(_KBLK, _KBLK)(x2, ids)
    o5a = _build_format(0)(g0)
    o5 = _build_format(1)(g1, o5a)
    return jnp.transpose(o5, (2, 4, 0, 1, 3)).reshape(_BATCH, _NCOLS * 4, 8, 8)


# speedup vs baseline: 1.1285x; 1.0381x over previous
import functools

import numpy as np
import jax
import jax.numpy as jnp
from jax import lax
from jax.experimental import pallas as pl
from jax.experimental.pallas import tpu as pltpu
from jax.experimental.pallas import tpu_sc as plsc

_SAMPLE_SIZE = 25
_NCOLS = _SAMPLE_SIZE + 1
_NODES = 50000
_BATCH = 4096
_ADJ_COLS = 65
_FEAT = 256
_NW = 32
_ROWS_PER_W = _BATCH // _NW
_IDS_PER_W = _ROWS_PER_W * _NCOLS
_CHUNK = 128
_NCHUNK = _IDS_PER_W // _CHUNK

_PERM25 = (35, 45, 31, 63, 7, 4, 29, 44, 16, 58, 37, 19, 61, 2, 34,
           5, 30, 42, 3, 39, 56, 22, 6, 54, 18)
_COLS = np.array([0] + [p + 1 for p in _PERM25], dtype=np.int32)

_cache = {}


def _flat_tab():
    if "tab" not in _cache:
        w, k, bi = np.meshgrid(np.arange(_NW), np.arange(_NCOLS),
                               np.arange(_ROWS_PER_W), indexing="ij")
        _cache["tab"] = (_COLS[k] * _BATCH
                         + (w * _ROWS_PER_W + bi)).reshape(-1).astype(np.int32)
    return _cache["tab"]


def _build_ids():
    if "ids" in _cache:
        return _cache["ids"]

    mesh = plsc.VectorSubcoreMesh(core_axis_name="c", subcore_axis_name="s")

    @functools.partial(
        pl.kernel,
        mesh=mesh,
        out_type=jax.ShapeDtypeStruct((_NW * _IDS_PER_W,), jnp.int32),
        scratch_types=[
            pltpu.VMEM((_IDS_PER_W,), jnp.int32),
            pltpu.VMEM((_IDS_PER_W,), jnp.int32),
            pltpu.SemaphoreType.DMA,
        ],
    )
    def _impl(adj_hbm, tab_hbm, out_hbm, tab_v, ids_v, asem):
        wid = lax.axis_index("s") * 2 + lax.axis_index("c")
        base = wid * _IDS_PER_W
        pltpu.sync_copy(tab_hbm.at[pl.ds(base, _IDS_PER_W)], tab_v)
        for c in range(_NCHUNK):
            sl = pl.ds(c * _CHUNK, _CHUNK)
            pltpu.make_async_copy(
                adj_hbm.at[tab_v.at[sl]], ids_v.at[sl], asem).start()
        pltpu.make_async_copy(adj_hbm.at[tab_v], ids_v, asem).wait()
        pltpu.sync_copy(ids_v, out_hbm.at[pl.ds(base, _IDS_PER_W)])

    _cache["ids"] = _impl
    return _impl


def _build_gather(k0, nk):
    key = ("call", k0, nk)
    if key in _cache:
        return _cache[key]

    ids_n = nk * _CHUNK
    mesh = plsc.VectorSubcoreMesh(core_axis_name="c", subcore_axis_name="s")

    @functools.partial(
        pl.kernel,
        mesh=mesh,
        out_type=jax.ShapeDtypeStruct((nk, _BATCH, _FEAT), jnp.float32),
        scratch_types=[
            pltpu.VMEM((ids_n,), jnp.int32),
            pltpu.VMEM((_CHUNK, _FEAT), jnp.float32),
            pltpu.VMEM((_CHUNK, _FEAT), jnp.float32),
            pltpu.VMEM((_CHUNK, _FEAT), jnp.float32),
            pltpu.SemaphoreType.DMA,
            pltpu.SemaphoreType.DMA,
            pltpu.SemaphoreType.DMA,
            pltpu.SemaphoreType.DMA,
            pltpu.SemaphoreType.DMA,
            pltpu.SemaphoreType.DMA,
        ],
    )
    def _impl(x_hbm, allids_hbm, out_hbm,
              ids_v, buf0, buf1, buf2,
              g0, g1, g2, w0, w1, w2):
        wid = lax.axis_index("s") * 2 + lax.axis_index("c")
        b0 = wid * _ROWS_PER_W

        pltpu.sync_copy(
            allids_hbm.at[pl.ds(wid * _IDS_PER_W + k0 * _CHUNK, ids_n)],
            ids_v)

        buf = (buf0, buf1, buf2)
        gsem = (g0, g1, g2)
        wsem = (w0, w1, w2)

        def g_copy(c, b):
            return pltpu.make_async_copy(
                x_hbm.at[ids_v.at[pl.ds(c * _CHUNK, _CHUNK)]],
                buf[b], gsem[b])

        def w_copy(c, b):
            return pltpu.make_async_copy(
                buf[b],
                out_hbm.at[c, pl.ds(b0, _ROWS_PER_W)], wsem[b])

        g_copy(0, 0).start()
        g_copy(1, 1).start()

        for c in range(nk):
            b = c % 3
            g_copy(c, b).wait()
            w_copy(c, b).start()
            if c + 2 < nk:
                if c >= 1:
                    w_copy(c - 1, (c - 1) % 3).wait()
                g_copy(c + 2, (c + 2) % 3).start()

        for c in range(max(nk - 3, 0), nk):
            w_copy(c, c % 3).wait()

    _cache[key] = _impl
    return _impl


_KBLK = 13


def _fmt_body(g_ref, o_ref):
    for kk in range(_KBLK):
        blk = g_ref[kk]
        o_ref[pl.ds(kk * 4, 4)] = jnp.swapaxes(blk, 0, 1).reshape(
            4, 8, 1, 8, _ROWS_PER_W)


def _fmt_body_alias(g_ref, o5in_ref, o_ref):
    del o5in_ref
    _fmt_body(g_ref, o_ref)


def _build_format(part):
    key = ("fmt", part)
    if key not in _cache:
        in_specs = [pl.BlockSpec((_KBLK, _ROWS_PER_W, _FEAT),
                                 lambda k, t: (k, t, 0))]
        kwargs = {}
        body = _fmt_body
        if part > 0:
            in_specs.append(pl.BlockSpec(memory_space=pl.ANY))
            kwargs["input_output_aliases"] = {1: 0}
            body = _fmt_body_alias
        _cache[key] = pl.pallas_call(
            body,
            grid=(1, _BATCH // _ROWS_PER_W),
            in_specs=in_specs,
            out_specs=pl.BlockSpec((4 * _KBLK, 8, 1, 8, _ROWS_PER_W),
                                   lambda k, t: (k + part, 0, t, 0, 0)),
            out_shape=jax.ShapeDtypeStruct(
                (_NCOLS * 4, 8, _BATCH // _ROWS_PER_W, 8, _ROWS_PER_W),
                jnp.float32),
            **kwargs,
        )
    return _cache[key]


_NBLK = 4096


def _xpose_body(xt_ref, x2_ref):
    x2_ref[...] = jnp.swapaxes(xt_ref[...], 0, 1)


def _build_xpose():
    if "xp" not in _cache:
        _cache["xp"] = pl.pallas_call(
            _xpose_body,
            grid=(pl.cdiv(_NODES, _NBLK),),
            in_specs=[pl.BlockSpec((_FEAT, _NBLK), lambda i: (0, i))],
            out_specs=pl.BlockSpec((_NBLK, _FEAT), lambda i: (i, 0)),
            out_shape=jax.ShapeDtypeStruct((_NODES, _FEAT), jnp.float32),
        )
    return _cache["xp"]


def kernel(x, adj_input):
    x2 = _build_xpose()(x.reshape(_NODES, _FEAT).T)
    adj = adj_input.astype(jnp.int32).T.reshape(_ADJ_COLS * _BATCH)
    tab = jnp.asarray(_flat_tab())
    ids = _build_ids()(adj, tab)
    g0 = _build_gather(0, _KBLK)(x2, ids)
    g1 = _build_gather(_KBLK, _KBLK)(x2, ids)
    o5a = _build_format(0)(g0)
    o5 = _build_format(1)(g1, o5a)
    return jnp.transpose(o5, (2, 4, 0, 1, 3)).reshape(_BATCH, _NCOLS * 4, 8, 8)


# speedup vs baseline: 1.1935x; 1.0576x over previous
import functools

import numpy as np
import jax
import jax.numpy as jnp
from jax import lax
from jax.experimental import pallas as pl
from jax.experimental.pallas import tpu as pltpu
from jax.experimental.pallas import tpu_sc as plsc

_SAMPLE_SIZE = 25
_NCOLS = _SAMPLE_SIZE + 1
_NODES = 50000
_BATCH = 4096
_ADJ_COLS = 65
_FEAT = 256
_NW = 32
_ROWS_PER_W = _BATCH // _NW
_IDS_PER_W = _ROWS_PER_W * _NCOLS
_CHUNK = 128
_NCHUNK = _IDS_PER_W // _CHUNK

_PERM25 = (35, 45, 31, 63, 7, 4, 29, 44, 16, 58, 37, 19, 61, 2, 34,
           5, 30, 42, 3, 39, 56, 22, 6, 54, 18)
_COLS = np.array([0] + [p + 1 for p in _PERM25], dtype=np.int32)

_cache = {}


def _flat_tab():
    if "tab" not in _cache:
        w, k, bi = np.meshgrid(np.arange(_NW), np.arange(_NCOLS),
                               np.arange(_ROWS_PER_W), indexing="ij")
        _cache["tab"] = (_COLS[k] * _BATCH
                         + (w * _ROWS_PER_W + bi)).reshape(-1).astype(np.int32)
    return _cache["tab"]


def _build_ids():
    if "ids" in _cache:
        return _cache["ids"]

    mesh = plsc.VectorSubcoreMesh(core_axis_name="c", subcore_axis_name="s")

    @functools.partial(
        pl.kernel,
        mesh=mesh,
        out_type=jax.ShapeDtypeStruct((_NW * _IDS_PER_W,), jnp.int32),
        scratch_types=[
            pltpu.VMEM((_IDS_PER_W,), jnp.int32),
            pltpu.VMEM((_IDS_PER_W,), jnp.int32),
            pltpu.SemaphoreType.DMA,
        ],
    )
    def _impl(adj_hbm, tab_hbm, out_hbm, tab_v, ids_v, asem):
        wid = lax.axis_index("s") * 2 + lax.axis_index("c")
        base = wid * _IDS_PER_W
        pltpu.sync_copy(tab_hbm.at[pl.ds(base, _IDS_PER_W)], tab_v)
        for c in range(_NCHUNK):
            sl = pl.ds(c * _CHUNK, _CHUNK)
            pltpu.make_async_copy(
                adj_hbm.at[tab_v.at[sl]], ids_v.at[sl], asem).start()
        pltpu.make_async_copy(adj_hbm.at[tab_v], ids_v, asem).wait()
        pltpu.sync_copy(ids_v, out_hbm.at[pl.ds(base, _IDS_PER_W)])

    _cache["ids"] = _impl
    return _impl


def _build_gather(k0, nk):
    key = ("call", k0, nk)
    if key in _cache:
        return _cache[key]

    ids_n = nk * _CHUNK
    mesh = plsc.VectorSubcoreMesh(core_axis_name="c", subcore_axis_name="s")

    @functools.partial(
        pl.kernel,
        mesh=mesh,
        out_type=jax.ShapeDtypeStruct((nk, _BATCH, _FEAT), jnp.float32),
        scratch_types=[
            pltpu.VMEM((ids_n,), jnp.int32),
            pltpu.VMEM((_CHUNK, _FEAT), jnp.float32),
            pltpu.VMEM((_CHUNK, _FEAT), jnp.float32),
            pltpu.VMEM((_CHUNK, _FEAT), jnp.float32),
            pltpu.SemaphoreType.DMA,
            pltpu.SemaphoreType.DMA,
            pltpu.SemaphoreType.DMA,
            pltpu.SemaphoreType.DMA,
            pltpu.SemaphoreType.DMA,
            pltpu.SemaphoreType.DMA,
        ],
    )
    def _impl(x_hbm, allids_hbm, out_hbm,
              ids_v, buf0, buf1, buf2,
              g0, g1, g2, w0, w1, w2):
        wid = lax.axis_index("s") * 2 + lax.axis_index("c")
        b0 = wid * _ROWS_PER_W

        pltpu.sync_copy(
            allids_hbm.at[pl.ds(wid * _IDS_PER_W + k0 * _CHUNK, ids_n)],
            ids_v)

        buf = (buf0, buf1, buf2)
        gsem = (g0, g1, g2)
        wsem = (w0, w1, w2)

        def g_copy(c, b):
            return pltpu.make_async_copy(
                x_hbm.at[ids_v.at[pl.ds(c * _CHUNK, _CHUNK)]],
                buf[b], gsem[b])

        def w_copy(c, b):
            return pltpu.make_async_copy(
                buf[b],
                out_hbm.at[c, pl.ds(b0, _ROWS_PER_W)], wsem[b])

        g_copy(0, 0).start()
        g_copy(1, 1).start()

        for c in range(nk):
            b = c % 3
            g_copy(c, b).wait()
            w_copy(c, b).start()
            if c + 2 < nk:
                if c >= 1:
                    w_copy(c - 1, (c - 1) % 3).wait()
                g_copy(c + 2, (c + 2) % 3).start()

        for c in range(max(nk - 3, 0), nk):
            w_copy(c, c % 3).wait()

    _cache[key] = _impl
    return _impl


_KBLK = 13
_TBLK = 4


def _make_fmt_body(nk):
    def body(*refs):
        g_ref, o_ref = refs[0], refs[-1]
        for kk in range(nk):
            for tt in range(_TBLK):
                blk = g_ref[kk, pl.ds(tt * _ROWS_PER_W, _ROWS_PER_W)]
                o_ref[pl.ds(kk * 4, 4), :, pl.ds(tt, 1)] = jnp.swapaxes(
                    blk, 0, 1).reshape(4, 8, 1, 8, _ROWS_PER_W)
    return body


def _build_format(part):
    key = ("fmt", part)
    if key not in _cache:
        in_specs = [pl.BlockSpec((_KBLK, _ROWS_PER_W * _TBLK, _FEAT),
                                 lambda k, t: (k, t, 0))]
        kwargs = {}
        if part > 0:
            in_specs.append(pl.BlockSpec(memory_space=pl.ANY))
            kwargs["input_output_aliases"] = {1: 0}
        _cache[key] = pl.pallas_call(
            _make_fmt_body(_KBLK),
            grid=(1, _BATCH // (_ROWS_PER_W * _TBLK)),
            in_specs=in_specs,
            out_specs=pl.BlockSpec((4 * _KBLK, 8, _TBLK, 8, _ROWS_PER_W),
                                   lambda k, t, part=part: (k + part, 0, t, 0, 0)),
            out_shape=jax.ShapeDtypeStruct(
                (_NCOLS * 4, 8, _BATCH // _ROWS_PER_W, 8, _ROWS_PER_W),
                jnp.float32),
            **kwargs,
        )
    return _cache[key]


_NBLK = 4096


def _xpose_body(xt_ref, x2_ref):
    x2_ref[...] = jnp.swapaxes(xt_ref[...], 0, 1)


def _build_xpose():
    if "xp" not in _cache:
        _cache["xp"] = pl.pallas_call(
            _xpose_body,
            grid=(pl.cdiv(_NODES, _NBLK),),
            in_specs=[pl.BlockSpec((_FEAT, _NBLK), lambda i: (0, i))],
            out_specs=pl.BlockSpec((_NBLK, _FEAT), lambda i: (i, 0)),
            out_shape=jax.ShapeDtypeStruct((_NODES, _FEAT), jnp.float32),
        )
    return _cache["xp"]


def kernel(x, adj_input):
    x2 = _build_xpose()(x.reshape(_NODES, _FEAT).T)
    adj = adj_input.astype(jnp.int32).T.reshape(_ADJ_COLS * _BATCH)
    tab = jnp.asarray(_flat_tab())
    ids = _build_ids()(adj, tab)
    g0 = _build_gather(0, _KBLK)(x2, ids)
    g1 = _build_gather(_KBLK, _KBLK)(x2, ids)
    o5a = _build_format(0)(g0)
    o5 = _build_format(1)(g1, o5a)
    return jnp.transpose(o5, (2, 4, 0, 1, 3)).reshape(_BATCH, _NCOLS * 4, 8, 8)


# speedup vs baseline: 1.1941x; 1.0005x over previous
import functools

import numpy as np
import jax
import jax.numpy as jnp
from jax import lax
from jax.experimental import pallas as pl
from jax.experimental.pallas import tpu as pltpu
from jax.experimental.pallas import tpu_sc as plsc

_SAMPLE_SIZE = 25
_NCOLS = _SAMPLE_SIZE + 1
_NODES = 50000
_BATCH = 4096
_ADJ_COLS = 65
_FEAT = 256
_NW = 32
_ROWS_PER_W = _BATCH // _NW
_IDS_PER_W = _ROWS_PER_W * _NCOLS
_CHUNK = 128
_NCHUNK = _IDS_PER_W // _CHUNK

_PERM25 = (35, 45, 31, 63, 7, 4, 29, 44, 16, 58, 37, 19, 61, 2, 34,
           5, 30, 42, 3, 39, 56, 22, 6, 54, 18)
_COLS = np.array([0] + [p + 1 for p in _PERM25], dtype=np.int32)

_cache = {}


def _flat_tab():
    if "tab" not in _cache:
        w, k, bi = np.meshgrid(np.arange(_NW), np.arange(_NCOLS),
                               np.arange(_ROWS_PER_W), indexing="ij")
        _cache["tab"] = (_COLS[k] * _BATCH
                         + (w * _ROWS_PER_W + bi)).reshape(-1).astype(np.int32)
    return _cache["tab"]


def _build_ids():
    if "ids" in _cache:
        return _cache["ids"]

    mesh = plsc.VectorSubcoreMesh(core_axis_name="c", subcore_axis_name="s")

    @functools.partial(
        pl.kernel,
        mesh=mesh,
        out_type=jax.ShapeDtypeStruct((_NW * _IDS_PER_W,), jnp.int32),
        scratch_types=[
            pltpu.VMEM((_IDS_PER_W,), jnp.int32),
            pltpu.VMEM((_IDS_PER_W,), jnp.int32),
            pltpu.SemaphoreType.DMA,
        ],
    )
    def _impl(adj_hbm, tab_hbm, out_hbm, tab_v, ids_v, asem):
        wid = lax.axis_index("s") * 2 + lax.axis_index("c")
        base = wid * _IDS_PER_W
        pltpu.sync_copy(tab_hbm.at[pl.ds(base, _IDS_PER_W)], tab_v)
        for c in range(_NCHUNK):
            sl = pl.ds(c * _CHUNK, _CHUNK)
            pltpu.make_async_copy(
                adj_hbm.at[tab_v.at[sl]], ids_v.at[sl], asem).start()
        pltpu.make_async_copy(adj_hbm.at[tab_v], ids_v, asem).wait()
        pltpu.sync_copy(ids_v, out_hbm.at[pl.ds(base, _IDS_PER_W)])

    _cache["ids"] = _impl
    return _impl


def _build_gather(k0, nk):
    key = ("call", k0, nk)
    if key in _cache:
        return _cache[key]

    ids_n = nk * _CHUNK
    mesh = plsc.VectorSubcoreMesh(core_axis_name="c", subcore_axis_name="s")

    @functools.partial(
        pl.kernel,
        mesh=mesh,
        out_type=jax.ShapeDtypeStruct((nk, _BATCH, _FEAT), jnp.float32),
        scratch_types=[
            pltpu.VMEM((ids_n,), jnp.int32),
            pltpu.VMEM((_CHUNK, _FEAT), jnp.float32),
            pltpu.VMEM((_CHUNK, _FEAT), jnp.float32),
            pltpu.VMEM((_CHUNK, _FEAT), jnp.float32),
            pltpu.SemaphoreType.DMA,
            pltpu.SemaphoreType.DMA,
            pltpu.SemaphoreType.DMA,
            pltpu.SemaphoreType.DMA,
            pltpu.SemaphoreType.DMA,
            pltpu.SemaphoreType.DMA,
        ],
    )
    def _impl(x_hbm, allids_hbm, out_hbm,
              ids_v, buf0, buf1, buf2,
              g0, g1, g2, w0, w1, w2):
        wid = lax.axis_index("s") * 2 + lax.axis_index("c")
        b0 = wid * _ROWS_PER_W

        pltpu.sync_copy(
            allids_hbm.at[pl.ds(wid * _IDS_PER_W + k0 * _CHUNK, ids_n)],
            ids_v)

        buf = (buf0, buf1, buf2)
        gsem = (g0, g1, g2)
        wsem = (w0, w1, w2)

        def g_copy(c, b):
            return pltpu.make_async_copy(
                x_hbm.at[ids_v.at[pl.ds(c * _CHUNK, _CHUNK)]],
                buf[b], gsem[b])

        def w_copy(c, b):
            return pltpu.make_async_copy(
                buf[b],
                out_hbm.at[c, pl.ds(b0, _ROWS_PER_W)], wsem[b])

        g_copy(0, 0).start()
        g_copy(1, 1).start()

        for c in range(nk):
            b = c % 3
            g_copy(c, b).wait()
            w_copy(c, b).start()
            if c + 2 < nk:
                if c >= 1:
                    w_copy(c - 1, (c - 1) % 3).wait()
                g_copy(c + 2, (c + 2) % 3).start()

        for c in range(max(nk - 3, 0), nk):
            w_copy(c, c % 3).wait()

    _cache[key] = _impl
    return _impl


_KBLK = 13
_TBLK = 8


def _make_fmt_body(nk):
    def body(*refs):
        g_ref, o_ref = refs[0], refs[-1]
        for kk in range(nk):
            for tt in range(_TBLK):
                blk = g_ref[kk, pl.ds(tt * _ROWS_PER_W, _ROWS_PER_W)]
                o_ref[pl.ds(kk * 4, 4), :, pl.ds(tt, 1)] = jnp.swapaxes(
                    blk, 0, 1).reshape(4, 8, 1, 8, _ROWS_PER_W)
    return body


def _build_format(part):
    key = ("fmt", part)
    if key not in _cache:
        in_specs = [pl.BlockSpec((_KBLK, _ROWS_PER_W * _TBLK, _FEAT),
                                 lambda k, t: (k, t, 0))]
        kwargs = {}
        if part > 0:
            in_specs.append(pl.BlockSpec(memory_space=pl.ANY))
            kwargs["input_output_aliases"] = {1: 0}
        _cache[key] = pl.pallas_call(
            _make_fmt_body(_KBLK),
            grid=(1, _BATCH // (_ROWS_PER_W * _TBLK)),
            in_specs=in_specs,
            out_specs=pl.BlockSpec((4 * _KBLK, 8, _TBLK, 8, _ROWS_PER_W),
                                   lambda k, t, part=part: (k + part, 0, t, 0, 0)),
            out_shape=jax.ShapeDtypeStruct(
                (_NCOLS * 4, 8, _BATCH // _ROWS_PER_W, 8, _ROWS_PER_W),
                jnp.float32),
            **kwargs,
        )
    return _cache[key]


_NBLK = 4096


def _xpose_body(xt_ref, x2_ref):
    x2_ref[...] = jnp.swapaxes(xt_ref[...], 0, 1)


def _build_xpose():
    if "xp" not in _cache:
        _cache["xp"] = pl.pallas_call(
            _xpose_body,
            grid=(pl.cdiv(_NODES, _NBLK),),
            in_specs=[pl.BlockSpec((_FEAT, _NBLK), lambda i: (0, i))],
            out_specs=pl.BlockSpec((_NBLK, _FEAT), lambda i: (i, 0)),
            out_shape=jax.ShapeDtypeStruct((_NODES, _FEAT), jnp.float32),
        )
    return _cache["xp"]


def kernel(x, adj_input):
    x2 = _build_xpose()(x.reshape(_NODES, _FEAT).T)
    adj = adj_input.astype(jnp.int32).T.reshape(_ADJ_COLS * _BATCH)
    tab = jnp.asarray(_flat_tab())
    ids = _build_ids()(adj, tab)
    g0 = _build_gather(0, _KBLK)(x2, ids)
    g1 = _build_gather(_KBLK, _KBLK)(x2, ids)
    o5a = _build_format(0)(g0)
    o5 = _build_format(1)(g1, o5a)
    return jnp.transpose(o5, (2, 4, 0, 1, 3)).reshape(_BATCH, _NCOLS * 4, 8, 8)


# speedup vs baseline: 1.2000x; 1.0050x over previous
import functools

import numpy as np
import jax
import jax.numpy as jnp
from jax import lax
from jax.experimental import pallas as pl
from jax.experimental.pallas import tpu as pltpu
from jax.experimental.pallas import tpu_sc as plsc

_SAMPLE_SIZE = 25
_NCOLS = _SAMPLE_SIZE + 1
_NODES = 50000
_BATCH = 4096
_ADJ_COLS = 65
_FEAT = 256
_NW = 32
_ROWS_PER_W = _BATCH // _NW
_IDS_PER_W = _ROWS_PER_W * _NCOLS
_CHUNK = 128
_NCHUNK = _IDS_PER_W // _CHUNK

_PERM25 = (35, 45, 31, 63, 7, 4, 29, 44, 16, 58, 37, 19, 61, 2, 34,
           5, 30, 42, 3, 39, 56, 22, 6, 54, 18)
_COLS = np.array([0] + [p + 1 for p in _PERM25], dtype=np.int32)

_cache = {}


def _flat_tab():
    if "tab" not in _cache:
        w, k, bi = np.meshgrid(np.arange(_NW), np.arange(_NCOLS),
                               np.arange(_ROWS_PER_W), indexing="ij")
        _cache["tab"] = (_COLS[k] * _BATCH
                         + (w * _ROWS_PER_W + bi)).reshape(-1).astype(np.int32)
    return _cache["tab"]


def _build_ids():
    if "ids" in _cache:
        return _cache["ids"]

    mesh = plsc.VectorSubcoreMesh(core_axis_name="c", subcore_axis_name="s")

    @functools.partial(
        pl.kernel,
        mesh=mesh,
        out_type=jax.ShapeDtypeStruct((_NW * _IDS_PER_W,), jnp.int32),
        scratch_types=[
            pltpu.VMEM((_IDS_PER_W,), jnp.int32),
            pltpu.VMEM((_IDS_PER_W,), jnp.int32),
            pltpu.SemaphoreType.DMA,
        ],
    )
    def _impl(adj_hbm, tab_hbm, out_hbm, tab_v, ids_v, asem):
        wid = lax.axis_index("s") * 2 + lax.axis_index("c")
        base = wid * _IDS_PER_W
        pltpu.sync_copy(tab_hbm.at[pl.ds(base, _IDS_PER_W)], tab_v)
        for c in range(_NCHUNK):
            sl = pl.ds(c * _CHUNK, _CHUNK)
            pltpu.make_async_copy(
                adj_hbm.at[tab_v.at[sl]], ids_v.at[sl], asem).start()
        pltpu.make_async_copy(adj_hbm.at[tab_v], ids_v, asem).wait()
        pltpu.sync_copy(ids_v, out_hbm.at[pl.ds(base, _IDS_PER_W)])

    _cache["ids"] = _impl
    return _impl


def _build_gather(k0, nk):
    key = ("call", k0, nk)
    if key in _cache:
        return _cache[key]

    ids_n = nk * _CHUNK
    mesh = plsc.VectorSubcoreMesh(core_axis_name="c", subcore_axis_name="s")

    @functools.partial(
        pl.kernel,
        mesh=mesh,
        out_type=jax.ShapeDtypeStruct((nk, _BATCH, _FEAT), jnp.float32),
        scratch_types=[
            pltpu.VMEM((ids_n,), jnp.int32),
            pltpu.VMEM((_CHUNK, _FEAT), jnp.float32),
            pltpu.VMEM((_CHUNK, _FEAT), jnp.float32),
            pltpu.VMEM((_CHUNK, _FEAT), jnp.float32),
            pltpu.SemaphoreType.DMA,
            pltpu.SemaphoreType.DMA,
            pltpu.SemaphoreType.DMA,
            pltpu.SemaphoreType.DMA,
            pltpu.SemaphoreType.DMA,
            pltpu.SemaphoreType.DMA,
        ],
    )
    def _impl(x_hbm, allids_hbm, out_hbm,
              ids_v, buf0, buf1, buf2,
              g0, g1, g2, w0, w1, w2):
        wid = lax.axis_index("s") * 2 + lax.axis_index("c")
        b0 = wid * _ROWS_PER_W

        pltpu.sync_copy(
            allids_hbm.at[pl.ds(wid * _IDS_PER_W + k0 * _CHUNK, ids_n)],
            ids_v)

        buf = (buf0, buf1, buf2)
        gsem = (g0, g1, g2)
        wsem = (w0, w1, w2)

        def g_copy(c, b):
            return pltpu.make_async_copy(
                x_hbm.at[ids_v.at[pl.ds(c * _CHUNK, _CHUNK)]],
                buf[b], gsem[b])

        def w_copy(c, b):
            return pltpu.make_async_copy(
                buf[b],
                out_hbm.at[c, pl.ds(b0, _ROWS_PER_W)], wsem[b])

        g_copy(0, 0).start()
        g_copy(1, 1).start()

        for c in range(nk):
            b = c % 3
            g_copy(c, b).wait()
            w_copy(c, b).start()
            if c + 2 < nk:
                if c >= 1:
                    w_copy(c - 1, (c - 1) % 3).wait()
                g_copy(c + 2, (c + 2) % 3).start()

        for c in range(max(nk - 3, 0), nk):
            w_copy(c, c % 3).wait()

    _cache[key] = _impl
    return _impl


_KBLK = 13
_TBLK = 4


def _make_fmt_body(nk):
    def body(*refs):
        g_ref, o_ref = refs[0], refs[-1]
        for kk in range(nk):
            for tt in range(_TBLK):
                blk = g_ref[kk, pl.ds(tt * _ROWS_PER_W, _ROWS_PER_W)]
                o_ref[pl.ds(kk * 4, 4), :, pl.ds(tt, 1)] = jnp.swapaxes(
                    blk, 0, 1).reshape(4, 8, 1, 8, _ROWS_PER_W)
    return body


def _build_format(part):
    key = ("fmt", part)
    if key not in _cache:
        in_specs = [pl.BlockSpec((_KBLK, _ROWS_PER_W * _TBLK, _FEAT),
                                 lambda k, t: (k, t, 0))]
        kwargs = {}
        if part > 0:
            in_specs.append(pl.BlockSpec(memory_space=pl.ANY))
            kwargs["input_output_aliases"] = {1: 0}
        _cache[key] = pl.pallas_call(
            _make_fmt_body(_KBLK),
            grid=(1, _BATCH // (_ROWS_PER_W * _TBLK)),
            in_specs=in_specs,
            out_specs=pl.BlockSpec((4 * _KBLK, 8, _TBLK, 8, _ROWS_PER_W),
                                   lambda k, t, part=part: (k + part, 0, t, 0, 0)),
            out_shape=jax.ShapeDtypeStruct(
                (_NCOLS * 4, 8, _BATCH // _ROWS_PER_W, 8, _ROWS_PER_W),
                jnp.float32),
            **kwargs,
        )
    return _cache[key]


_NBLK = 8192


def _xpose_body(xt_ref, x2_ref):
    x2_ref[...] = jnp.swapaxes(xt_ref[...], 0, 1)


def _build_xpose():
    if "xp" not in _cache:
        _cache["xp"] = pl.pallas_call(
            _xpose_body,
            grid=(pl.cdiv(_NODES, _NBLK),),
            in_specs=[pl.BlockSpec((_FEAT, _NBLK), lambda i: (0, i))],
            out_specs=pl.BlockSpec((_NBLK, _FEAT), lambda i: (i, 0)),
            out_shape=jax.ShapeDtypeStruct((_NODES, _FEAT), jnp.float32),
        )
    return _cache["xp"]


def kernel(x, adj_input):
    x2 = _build_xpose()(x.reshape(_NODES, _FEAT).T)
    adj = adj_input.astype(jnp.int32).T.reshape(_ADJ_COLS * _BATCH)
    tab = jnp.asarray(_flat_tab())
    ids = _build_ids()(adj, tab)
    g0 = _build_gather(0, _KBLK)(x2, ids)
    g1 = _build_gather(_KBLK, _KBLK)(x2, ids)
    o5a = _build_format(0)(g0)
    o5 = _build_format(1)(g1, o5a)
    return jnp.transpose(o5, (2, 4, 0, 1, 3)).reshape(_BATCH, _NCOLS * 4, 8, 8)
